# baseline (speedup 1.0000x reference)
import functools

import jax
import jax.numpy as jnp
from jax import lax
from jax.experimental import pallas as pl
from jax.experimental.pallas import tpu as pltpu
from jax.experimental.pallas import tpu_sc as plsc

_B = 10000
_W = 128


def _dotb(a, b, dims):
    return lax.dot_general(a.astype(jnp.bfloat16), b.astype(jnp.bfloat16),
                           (dims, ((), ())),
                           preferred_element_type=jnp.float32)


def _u1p_body(u_ref, w1u_ref, out_ref):
    out_ref[...] = _dotb(u_ref[...], w1u_ref[...], ((1,), (1,)))


def _sc_gather(table, idx, np_rows, d):
    info = plsc.get_sparse_core_info()
    nc, ns = info.num_cores, info.num_subcores
    nw = nc * ns
    b_per_w = np_rows // nw
    n_chunks = 8
    c = b_per_w // n_chunks
    mesh = plsc.VectorSubcoreMesh(core_axis_name="c", subcore_axis_name="s")

    @functools.partial(
        pl.kernel, mesh=mesh,
        out_type=jax.ShapeDtypeStruct((np_rows, d), jnp.float32),
        scratch_types=[
            pltpu.VMEM((c,), jnp.int32),
            pltpu.VMEM((c, d), jnp.float32),
            pltpu.SemaphoreType.DMA,
        ],
    )
    def k(table_hbm, idx_hbm, out_hbm, idx_v, rows_v, sem):
        wid = lax.axis_index("s") * nc + lax.axis_index("c")
        for j in range(n_chunks):
            base = wid * b_per_w + j * c
            pltpu.sync_copy(idx_hbm.at[pl.ds(base, c)], idx_v)
            pltpu.async_copy(table_hbm.at[idx_v], rows_v, sem).wait()
            pltpu.sync_copy(rows_v, out_hbm.at[pl.ds(base, c)])

    return k(table, idx)


def _body(lohi_ref, x_ref, batch_ref, ug_ref, u_ref, w1x_ref, b1_ref,
          w2_ref, b2_ref, w3_ref, b3_ref, g_ref, beta_ref,
          pa_ref, pu_ref, pb1_ref, pw2_ref, pb2_ref, pw3_ref, pb3_ref,
          pg_ref, pbeta_ref, out_ref, acc_ref, *, nb, G, DH):
    i = pl.program_id(0)

    @pl.when(i == 0)
    def _init():
        acc_ref[...] = jnp.zeros_like(acc_ref)

    ids = batch_ref[0]
    lo = lohi_ref[i, 0]
    hi = lohi_ref[i, 1]
    nwin = (hi - lo) // _W + 1
    iota_w = lax.broadcasted_iota(jnp.int32, (_W, 1), 0)

    h = _dotb(x_ref[...], w1x_ref[...], ((1,), (1,))) + ug_ref[...]
    h = jnp.maximum(h + b1_ref[...], 0.0)
    h = _dotb(h, w2_ref[...], ((1,), (1,)))
    h = jnp.maximum(h + b2_ref[...], 0.0)
    h = _dotb(h, w3_ref[...], ((1,), (1,))) + b3_ref[...]
    mu = jnp.mean(h, axis=-1, keepdims=True)
    var = jnp.mean(jnp.square(h - mu), axis=-1, keepdims=True)
    h = (h - mu) * lax.rsqrt(var + 1e-5) * g_ref[...] + beta_ref[...]

    @pl.when(nwin == 1)
    def _single_window():
        oh_t = (ids - lo == iota_w).astype(jnp.bfloat16)
        acc_ref[pl.ds(lo, _W), :] += _dotb(oh_t, h, ((1,), (0,)))

    @pl.when(nwin != 1)
    def _multi_window():
        def _scatter_step(w, carry):
            base = lo + w * _W
            oh_t = (ids - base == iota_w).astype(jnp.bfloat16)
            acc_ref[pl.ds(base, _W), :] += _dotb(oh_t, h, ((1,), (0,)))
            return carry

        lax.fori_loop(0, nwin, _scatter_step, 0)

    @pl.when(i == nb - 1)
    def _post():
        agg = acc_ref[pl.ds(0, G), :]
        uu = u_ref[...]
        q = _dotb(agg, pa_ref[...], ((1,), (1,)))
        q += _dotb(uu, pu_ref[...], ((1,), (1,)))
        q = jnp.maximum(q + pb1_ref[...], 0.0)
        q = _dotb(q, pw2_ref[...], ((1,), (1,)))
        q = jnp.maximum(q + pb2_ref[...], 0.0)
        q = _dotb(q, pw3_ref[...], ((1,), (1,))) + pb3_ref[...]
        mu2 = jnp.mean(q, axis=-1, keepdims=True)
        var2 = jnp.mean(jnp.square(q - mu2), axis=-1, keepdims=True)
        q = (q - mu2) * lax.rsqrt(var2 + 1e-5) * pg_ref[...] + pbeta_ref[...]
        out_ref[...] = q + uu


def kernel(x, u, batch, pre_W1, pre_b1, pre_W2, pre_b2, pre_W3, pre_b3,
           pre_g, pre_beta, post_W1, post_b1, post_W2, post_b2, post_W3,
           post_b3, post_g, post_beta):
    N, DL = x.shape
    G, DG = u.shape
    DH = pre_W2.shape[0]
    DP = pre_W3.shape[0]
    nb = N // _B

    batch = batch.astype(jnp.int32)
    b2d = batch.reshape(nb, _B)
    lohi = jnp.stack([b2d[:, 0], b2d[:, -1]], axis=1)
    batch3d = batch.reshape(nb, 1, _B)

    w1x = pre_W1[:, :DL]
    w1u = pre_W1[:, DL:]
    pa = post_W1[:, :DP]
    pu = post_W1[:, DP:]
    row = lambda v: v.reshape(1, -1)

    u1p = pl.pallas_call(
        _u1p_body,
        out_shape=jax.ShapeDtypeStruct((G, DH), jnp.float32),
    )(u, w1u)

    np_rows = 100352
    idx_pad = jnp.pad(batch, (0, np_rows - N))
    ug = _sc_gather(u1p, idx_pad, np_rows, DH)

    full = lambda s: pl.BlockSpec(s, lambda i, sref: tuple(0 for _ in s))
    grid_spec = pltpu.PrefetchScalarGridSpec(
        num_scalar_prefetch=1,
        grid=(nb,),
        in_specs=[
            pl.BlockSpec((_B, DL), lambda i, sref: (i, 0)),
            pl.BlockSpec((1, 1, _B), lambda i, sref: (i, 0, 0)),
            pl.BlockSpec((_B, DH), lambda i, sref: (i, 0)),
            full((G, DG)),
            full((DH, DL)), full((1, DH)),
            full((DH, DH)), full((1, DH)),
            full((DP, DH)), full((1, DP)),
            full((1, DP)), full((1, DP)),
            full((DH, DP)), full((DH, DG)), full((1, DH)),
            full((DH, DH)), full((1, DH)),
            full((DG, DH)), full((1, DG)),
            full((1, DG)), full((1, DG)),
        ],
        out_specs=pl.BlockSpec((G, DG), lambda i, sref: (0, 0)),
        scratch_shapes=[
            pltpu.VMEM((G + _W, DH), jnp.float32),
        ],
    )

    body = functools.partial(_body, nb=nb, G=G, DH=DH)
    return pl.pallas_call(
        body,
        grid_spec=grid_spec,
        out_shape=jax.ShapeDtypeStruct((G, DG), jnp.float32),
        compiler_params=pltpu.CompilerParams(
            dimension_semantics=("arbitrary",)),
    )(lohi, x, batch3d, ug, u,
      w1x, row(pre_b1), pre_W2, row(pre_b2), pre_W3, row(pre_b3),
      row(pre_g), row(pre_beta),
      pa, pu, row(post_b1), post_W2, row(post_b2), post_W3, row(post_b3),
      row(post_g), row(post_beta))


# speedup vs baseline: 3.3703x; 3.3703x over previous
import functools

import jax
import jax.numpy as jnp
from jax import lax
from jax.experimental import pallas as pl
from jax.experimental.pallas import tpu as pltpu

_B = 10000
_W = 128


def _dotb(a, b, dims):
    return lax.dot_general(a.astype(jnp.bfloat16), b.astype(jnp.bfloat16),
                           (dims, ((), ())),
                           preferred_element_type=jnp.float32)


def _body(lohi_ref, x_ref, batch_ref, u_ref, w1x_ref, w1u_ref, b1_ref,
          w2_ref, b2_ref, w3_ref, b3_ref, g_ref, beta_ref,
          pa_ref, pu_ref, pb1_ref, pw2_ref, pb2_ref, pw3_ref, pb3_ref,
          pg_ref, pbeta_ref, out_ref, u1p_ref, acc_ref, *, nb, G, DH):
    i = pl.program_id(0)
    f32 = jnp.float32

    @pl.when(i == 0)
    def _init():
        u1p_ref[pl.ds(0, G), :] = _dotb(u_ref[...], w1u_ref[...],
                                        ((1,), (1,)))
        u1p_ref[pl.ds(G, _W), :] = jnp.zeros((_W, DH), f32)
        acc_ref[...] = jnp.zeros_like(acc_ref)

    ids = batch_ref[0]
    lo = lohi_ref[i, 0]
    hi = lohi_ref[i, 1]
    nwin = (hi - lo) // _W + 1
    iota_w = lax.broadcasted_iota(jnp.int32, (_W, 1), 0)

    def _mlp(pre1):
        h = jnp.maximum(pre1 + b1_ref[...], 0.0)
        h = _dotb(h, w2_ref[...], ((1,), (1,)))
        h = jnp.maximum(h + b2_ref[...], 0.0)
        h = _dotb(h, w3_ref[...], ((1,), (1,))) + b3_ref[...]
        mu = jnp.mean(h, axis=-1, keepdims=True)
        var = jnp.mean(jnp.square(h - mu), axis=-1, keepdims=True)
        return (h - mu) * lax.rsqrt(var + 1e-5) * g_ref[...] + beta_ref[...]

    @pl.when(nwin == 1)
    def _single_window():
        oh_t = (ids - lo == iota_w).astype(jnp.bfloat16)
        win = u1p_ref[pl.ds(lo, _W), :]
        h = _mlp(_dotb(x_ref[...], w1x_ref[...], ((1,), (1,)))
                 + _dotb(oh_t, win, ((0,), (0,))))
        acc_ref[pl.ds(lo, _W), :] += _dotb(oh_t, h, ((1,), (0,)))

    @pl.when(nwin != 1)
    def _multi_window():
        def _onehot_t(w):
            base = lo + w * _W
            return base, (ids - base == iota_w).astype(f32)

        def _gather_step(w, carry):
            base, oh_t = _onehot_t(w)
            win = u1p_ref[pl.ds(base, _W), :]
            return carry + _dotb(oh_t, win, ((0,), (0,)))

        gathered = lax.fori_loop(0, nwin, _gather_step,
                                 jnp.zeros((_B, DH), f32))
        h = _mlp(_dotb(x_ref[...], w1x_ref[...], ((1,), (1,))) + gathered)

        def _scatter_step(w, carry):
            base, oh_t = _onehot_t(w)
            acc_ref[pl.ds(base, _W), :] += _dotb(oh_t, h, ((1,), (0,)))
            return carry

        lax.fori_loop(0, nwin, _scatter_step, 0)

    @pl.when(i == nb - 1)
    def _post():
        agg = acc_ref[pl.ds(0, G), :]
        uu = u_ref[...]
        q = _dotb(agg, pa_ref[...], ((1,), (1,)))
        q += _dotb(uu, pu_ref[...], ((1,), (1,)))
        q = jnp.maximum(q + pb1_ref[...], 0.0)
        q = _dotb(q, pw2_ref[...], ((1,), (1,)))
        q = jnp.maximum(q + pb2_ref[...], 0.0)
        q = _dotb(q, pw3_ref[...], ((1,), (1,))) + pb3_ref[...]
        mu2 = jnp.mean(q, axis=-1, keepdims=True)
        var2 = jnp.mean(jnp.square(q - mu2), axis=-1, keepdims=True)
        q = (q - mu2) * lax.rsqrt(var2 + 1e-5) * pg_ref[...] + pbeta_ref[...]
        out_ref[...] = q + uu


def kernel(x, u, batch, pre_W1, pre_b1, pre_W2, pre_b2, pre_W3, pre_b3,
           pre_g, pre_beta, post_W1, post_b1, post_W2, post_b2, post_W3,
           post_b3, post_g, post_beta):
    N, DL = x.shape
    G, DG = u.shape
    DH = pre_W2.shape[0]
    DP = pre_W3.shape[0]
    nb = N // _B

    batch = batch.astype(jnp.int32)
    b2d = batch.reshape(nb, _B)
    lohi = jnp.stack([b2d[:, 0], b2d[:, -1]], axis=1)
    batch3d = batch.reshape(nb, 1, _B)

    w1x = pre_W1[:, :DL]
    w1u = pre_W1[:, DL:]
    pa = post_W1[:, :DP]
    pu = post_W1[:, DP:]
    row = lambda v: v.reshape(1, -1)

    full = lambda s: pl.BlockSpec(s, lambda i, sref: tuple(0 for _ in s))
    grid_spec = pltpu.PrefetchScalarGridSpec(
        num_scalar_prefetch=1,
        grid=(nb,),
        in_specs=[
            pl.BlockSpec((_B, DL), lambda i, sref: (i, 0)),
            pl.BlockSpec((1, 1, _B), lambda i, sref: (i, 0, 0)),
            full((G, DG)),
            full((DH, DL)), full((DH, DG)), full((1, DH)),
            full((DH, DH)), full((1, DH)),
            full((DP, DH)), full((1, DP)),
            full((1, DP)), full((1, DP)),
            full((DH, DP)), full((DH, DG)), full((1, DH)),
            full((DH, DH)), full((1, DH)),
            full((DG, DH)), full((1, DG)),
            full((1, DG)), full((1, DG)),
        ],
        out_specs=pl.BlockSpec((G, DG), lambda i, sref: (0, 0)),
        scratch_shapes=[
            pltpu.VMEM((G + _W, DH), jnp.float32),
            pltpu.VMEM((G + _W, DH), jnp.float32),
        ],
    )

    body = functools.partial(_body, nb=nb, G=G, DH=DH)
    return pl.pallas_call(
        body,
        grid_spec=grid_spec,
        out_shape=jax.ShapeDtypeStruct((G, DG), jnp.float32),
        compiler_params=pltpu.CompilerParams(
            dimension_semantics=("arbitrary",)),
    )(lohi, x, batch3d, u,
      w1x, w1u, row(pre_b1), pre_W2, row(pre_b2), pre_W3, row(pre_b3),
      row(pre_g), row(pre_beta),
      pa, pu, row(post_b1), post_W2, row(post_b2), post_W3, row(post_b3),
      row(post_g), row(post_beta))


# speedup vs baseline: 3.6044x; 1.0694x over previous
import functools

import jax
import jax.numpy as jnp
from jax import lax
from jax.experimental import pallas as pl
from jax.experimental.pallas import tpu as pltpu

_B = 20000
_W = 256


def _dotb(a, b, dims):
    return lax.dot_general(a.astype(jnp.bfloat16), b.astype(jnp.bfloat16),
                           (dims, ((), ())),
                           preferred_element_type=jnp.float32)


def _body(lohi_ref, x_ref, batch_ref, u_ref, w1x_ref, w1u_ref, b1_ref,
          w2_ref, b2_ref, w3_ref, b3_ref, g_ref, beta_ref,
          pa_ref, pu_ref, pb1_ref, pw2_ref, pb2_ref, pw3_ref, pb3_ref,
          pg_ref, pbeta_ref, out_ref, u1p_ref, acc_ref, *, nb, G, DH):
    i = pl.program_id(0)
    f32 = jnp.float32

    @pl.when(i == 0)
    def _init():
        u1p_ref[pl.ds(0, G), :] = _dotb(u_ref[...], w1u_ref[...],
                                        ((1,), (1,)))
        u1p_ref[pl.ds(G, _W), :] = jnp.zeros((_W, DH), f32)
        acc_ref[...] = jnp.zeros_like(acc_ref)

    ids = batch_ref[0]
    lo = lohi_ref[i, 0]
    hi = lohi_ref[i, 1]
    nwin = (hi - lo) // _W + 1
    iota_w = lax.broadcasted_iota(jnp.int32, (_W, 1), 0)

    def _mlp(pre1):
        h = jnp.maximum(pre1 + b1_ref[...], 0.0)
        h = _dotb(h, w2_ref[...], ((1,), (1,)))
        h = jnp.maximum(h + b2_ref[...], 0.0)
        h = _dotb(h, w3_ref[...], ((1,), (1,))) + b3_ref[...]
        mu = jnp.mean(h, axis=-1, keepdims=True)
        var = jnp.mean(jnp.square(h - mu), axis=-1, keepdims=True)
        return (h - mu) * lax.rsqrt(var + 1e-5) * g_ref[...] + beta_ref[...]

    @pl.when(nwin == 1)
    def _single_window():
        oh_t = (ids - lo == iota_w).astype(jnp.bfloat16)
        win = u1p_ref[pl.ds(lo, _W), :]
        h = _mlp(_dotb(x_ref[...], w1x_ref[...], ((1,), (1,)))
                 + _dotb(oh_t, win, ((0,), (0,))))
        acc_ref[pl.ds(lo, _W), :] += _dotb(oh_t, h, ((1,), (0,)))

    @pl.when(nwin != 1)
    def _multi_window():
        def _onehot_t(w):
            base = lo + w * _W
            return base, (ids - base == iota_w).astype(f32)

        def _gather_step(w, carry):
            base, oh_t = _onehot_t(w)
            win = u1p_ref[pl.ds(base, _W), :]
            return carry + _dotb(oh_t, win, ((0,), (0,)))

        gathered = lax.fori_loop(0, nwin, _gather_step,
                                 jnp.zeros((_B, DH), f32))
        h = _mlp(_dotb(x_ref[...], w1x_ref[...], ((1,), (1,))) + gathered)

        def _scatter_step(w, carry):
            base, oh_t = _onehot_t(w)
            acc_ref[pl.ds(base, _W), :] += _dotb(oh_t, h, ((1,), (0,)))
            return carry

        lax.fori_loop(0, nwin, _scatter_step, 0)

    @pl.when(i == nb - 1)
    def _post():
        agg = acc_ref[pl.ds(0, G), :]
        uu = u_ref[...]
        q = _dotb(agg, pa_ref[...], ((1,), (1,)))
        q += _dotb(uu, pu_ref[...], ((1,), (1,)))
        q = jnp.maximum(q + pb1_ref[...], 0.0)
        q = _dotb(q, pw2_ref[...], ((1,), (1,)))
        q = jnp.maximum(q + pb2_ref[...], 0.0)
        q = _dotb(q, pw3_ref[...], ((1,), (1,))) + pb3_ref[...]
        mu2 = jnp.mean(q, axis=-1, keepdims=True)
        var2 = jnp.mean(jnp.square(q - mu2), axis=-1, keepdims=True)
        q = (q - mu2) * lax.rsqrt(var2 + 1e-5) * pg_ref[...] + pbeta_ref[...]
        out_ref[...] = q + uu


def kernel(x, u, batch, pre_W1, pre_b1, pre_W2, pre_b2, pre_W3, pre_b3,
           pre_g, pre_beta, post_W1, post_b1, post_W2, post_b2, post_W3,
           post_b3, post_g, post_beta):
    N, DL = x.shape
    G, DG = u.shape
    DH = pre_W2.shape[0]
    DP = pre_W3.shape[0]
    nb = N // _B

    batch = batch.astype(jnp.int32)
    b2d = batch.reshape(nb, _B)
    lohi = jnp.stack([b2d[:, 0], b2d[:, -1]], axis=1)
    batch3d = batch.reshape(nb, 1, _B)

    w1x = pre_W1[:, :DL]
    w1u = pre_W1[:, DL:]
    pa = post_W1[:, :DP]
    pu = post_W1[:, DP:]
    row = lambda v: v.reshape(1, -1)

    full = lambda s: pl.BlockSpec(s, lambda i, sref: tuple(0 for _ in s))
    grid_spec = pltpu.PrefetchScalarGridSpec(
        num_scalar_prefetch=1,
        grid=(nb,),
        in_specs=[
            pl.BlockSpec((_B, DL), lambda i, sref: (i, 0)),
            pl.BlockSpec((1, 1, _B), lambda i, sref: (i, 0, 0)),
            full((G, DG)),
            full((DH, DL)), full((DH, DG)), full((1, DH)),
            full((DH, DH)), full((1, DH)),
            full((DP, DH)), full((1, DP)),
            full((1, DP)), full((1, DP)),
            full((DH, DP)), full((DH, DG)), full((1, DH)),
            full((DH, DH)), full((1, DH)),
            full((DG, DH)), full((1, DG)),
            full((1, DG)), full((1, DG)),
        ],
        out_specs=pl.BlockSpec((G, DG), lambda i, sref: (0, 0)),
        scratch_shapes=[
            pltpu.VMEM((G + _W, DH), jnp.float32),
            pltpu.VMEM((G + _W, DH), jnp.float32),
        ],
    )

    body = functools.partial(_body, nb=nb, G=G, DH=DH)
    return pl.pallas_call(
        body,
        grid_spec=grid_spec,
        out_shape=jax.ShapeDtypeStruct((G, DG), jnp.float32),
        compiler_params=pltpu.CompilerParams(
            dimension_semantics=("arbitrary",)),
    )(lohi, x, batch3d, u,
      w1x, w1u, row(pre_b1), pre_W2, row(pre_b2), pre_W3, row(pre_b3),
      row(pre_g), row(pre_beta),
      pa, pu, row(post_b1), post_W2, row(post_b2), post_W3, row(post_b3),
      row(post_g), row(post_beta))
